# baseline (speedup 1.0000x reference)
import jax
import jax.numpy as jnp
from jax import lax
from jax.experimental import pallas as pl
from jax.experimental.pallas import tpu as pltpu
from jax.experimental.pallas import tpu_sc as plsc

N = 10000
E = 320000
D = 128

NC = 2
NS = 16
NW = NC * NS

CHUNK = 128
EPAD = 327680
CPT = EPAD // (NW * CHUNK)
IB = 16
NBLK = CPT // IB
NBUF = 2
WB = 128
NPAD = 10240
RPT = NPAD // NS


def _sc_body(feat_h, edges_h, agg_h, deg_h,
             srcb, dstb, rows, zvec, onesv, aggsh, degsh,
             sem_g, sem_s, sem_d, sem_i):
    c = lax.axis_index("c")
    s = lax.axis_index("s")
    wid = c * NS + s
    zero16 = jnp.zeros((16,), jnp.float32)
    one16 = jnp.ones((16,), jnp.float32)

    for i in range(CHUNK):
        for j in range(D // 16):
            rows[0, i, pl.ds(j * 16, 16)] = zero16
    for j in range(RPT // 16):
        zvec[pl.ds(j * 16, 16)] = zero16
    for j in range(CHUNK // 16):
        onesv[pl.ds(j * 16, 16)] = one16

    for k in range(RPT // WB):
        pltpu.sync_copy(rows.at[0], aggsh.at[pl.ds(s * RPT + k * WB, WB)])
    pltpu.sync_copy(zvec, degsh.at[pl.ds(s * RPT, RPT)])
    plsc.subcore_barrier()

    pltpu.sync_copy(edges_h.at[0, pl.ds(wid * CPT, IB)], srcb.at[0])
    pltpu.sync_copy(edges_h.at[1, pl.ds(wid * CPT, IB)], dstb.at[0])

    def blk_body(blk, carry):
        sl = blk % 2
        nsl = (blk + 1) % 2
        nbase = wid * CPT + (blk + 1) * IB

        @pl.when(blk + 1 < NBLK)
        def _prefetch():
            pltpu.async_copy(edges_h.at[0, pl.ds(nbase, IB)], srcb.at[nsl],
                             sem_i)
            pltpu.async_copy(edges_h.at[1, pl.ds(nbase, IB)], dstb.at[nsl],
                             sem_i)

        src = srcb.at[sl]
        dst = dstb.at[sl]
        g = [None] * IB
        sc = [None] * IB
        dg = [None] * IB
        g[0] = pltpu.async_copy(feat_h.at[src.at[0]], rows.at[0], sem_g)
        for j in range(IB):
            b = j % NBUF
            nb = (j + 1) % NBUF
            if j + 1 < IB:
                if j + 1 >= NBUF:
                    sc[j + 1 - NBUF].wait()
                g[j + 1] = pltpu.async_copy(feat_h.at[src.at[j + 1]],
                                            rows.at[nb], sem_g)
            g[j].wait()
            sc[j] = pltpu.async_copy(rows.at[b], aggsh.at[dst.at[j]],
                                     sem_s, add=True)
            dg[j] = pltpu.async_copy(onesv, degsh.at[dst.at[j]],
                                     sem_d, add=True)
        for j in range(IB - NBUF, IB):
            sc[j].wait()
        for j in range(IB):
            dg[j].wait()

        @pl.when(blk + 1 < NBLK)
        def _drain_prefetch():
            pltpu.make_async_copy(edges_h.at[0, pl.ds(nbase, IB)],
                                  srcb.at[nsl], sem_i).wait()
            pltpu.make_async_copy(edges_h.at[1, pl.ds(nbase, IB)],
                                  dstb.at[nsl], sem_i).wait()

        return carry

    lax.fori_loop(0, NBLK, blk_body, 0)
    plsc.subcore_barrier()

    for k in range(RPT // WB):
        base = s * RPT + k * WB
        pltpu.sync_copy(aggsh.at[pl.ds(base, WB)], rows.at[0])
        pltpu.sync_copy(rows.at[0], agg_h.at[c, pl.ds(base, WB)])
    pltpu.sync_copy(degsh.at[pl.ds(s * RPT, RPT)], zvec)
    pltpu.sync_copy(zvec, deg_h.at[c, pl.ds(s * RPT, RPT)])


_sc_call = pl.kernel(
    _sc_body,
    mesh=plsc.VectorSubcoreMesh(core_axis_name="c", subcore_axis_name="s",
                                num_cores=NC),
    out_type=[
        jax.ShapeDtypeStruct((NC, NPAD, D), jnp.float32),
        jax.ShapeDtypeStruct((NC, NPAD), jnp.float32),
    ],
    scratch_types=[
        pltpu.VMEM((2, IB, CHUNK), jnp.int32),
        pltpu.VMEM((2, IB, CHUNK), jnp.int32),
        pltpu.VMEM((NBUF, CHUNK, D), jnp.float32),
        pltpu.VMEM((RPT,), jnp.float32),
        pltpu.VMEM((CHUNK,), jnp.float32),
        pltpu.VMEM_SHARED((NPAD, D), jnp.float32),
        pltpu.VMEM_SHARED((NPAD,), jnp.float32),
        pltpu.SemaphoreType.DMA,
        pltpu.SemaphoreType.DMA,
        pltpu.SemaphoreType.DMA,
        pltpu.SemaphoreType.DMA,
    ],
)


BN = 2560


def _tc_body(a_ref, d_ref, w_ref, b_ref, o_ref):
    agg = a_ref[0]
    deg = d_ref[0]
    for c in range(1, NC):
        agg = agg + a_ref[c]
        deg = deg + d_ref[c]
    deg = jnp.maximum(deg, 1.0)
    rst = agg / deg[:, None]
    o_ref[...] = jnp.dot(rst, w_ref[...],
                         preferred_element_type=jnp.float32) + b_ref[...]


_tc_call = pl.pallas_call(
    _tc_body,
    grid=(NPAD // BN,),
    in_specs=[
        pl.BlockSpec((NC, BN, D), lambda i: (0, i, 0)),
        pl.BlockSpec((NC, BN), lambda i: (0, i)),
        pl.BlockSpec((D, D), lambda i: (0, 0)),
        pl.BlockSpec((1, D), lambda i: (0, 0)),
    ],
    out_specs=pl.BlockSpec((BN, D), lambda i: (i, 0)),
    out_shape=jax.ShapeDtypeStruct((N, D), jnp.float32),
)


def kernel(feat, edge_index, W, b):
    pad = EPAD - E
    ar = jnp.arange(pad, dtype=jnp.int32)
    pad_src = ar % N
    pad_dst = N + (ar % (NPAD - N))
    ei = jnp.concatenate(
        [edge_index.astype(jnp.int32), jnp.stack([pad_src, pad_dst])], axis=1)
    ei = ei.reshape(2, EPAD // CHUNK, CHUNK)
    agg2, degp = _sc_call(feat, ei)
    return _tc_call(agg2, degp, W, b.reshape(1, D))


# speedup vs baseline: 1.0193x; 1.0193x over previous
import jax
import jax.numpy as jnp
from jax import lax
from jax.experimental import pallas as pl
from jax.experimental.pallas import tpu as pltpu
from jax.experimental.pallas import tpu_sc as plsc

N = 10000
E = 320000
D = 128

NC = 2
NS = 16
NW = NC * NS

CHUNK = 128
EPAD = 327680
CPT = EPAD // (NW * CHUNK)
IB = 16
NBLK = CPT // IB
NBUF = 2
WB = 128
NPAD = 10240
RPT = NPAD // NS


def _sc_body(feat_h, edges_h, agg_h, deg_h,
             srcb, dstb, rows, zvec, onesv, aggsh, degsh,
             sem_g, sem_s, sem_d, sem_i):
    c = lax.axis_index("c")
    s = lax.axis_index("s")
    wid = c * NS + s
    zero16 = jnp.zeros((16,), jnp.float32)
    one16 = jnp.ones((16,), jnp.float32)

    for i in range(CHUNK):
        for j in range(D // 16):
            rows[0, i, pl.ds(j * 16, 16)] = zero16
    for j in range(RPT // 16):
        zvec[pl.ds(j * 16, 16)] = zero16
    for j in range(CHUNK // 16):
        onesv[pl.ds(j * 16, 16)] = one16

    i0 = pltpu.async_copy(edges_h.at[0, pl.ds(wid * CPT, IB)], srcb.at[0],
                          sem_i)
    i1 = pltpu.async_copy(edges_h.at[1, pl.ds(wid * CPT, IB)], dstb.at[0],
                          sem_i)
    for k in range(RPT // WB):
        pltpu.sync_copy(rows.at[0], aggsh.at[pl.ds(s * RPT + k * WB, WB)])
    pltpu.sync_copy(zvec, degsh.at[pl.ds(s * RPT, RPT)])
    plsc.subcore_barrier()
    i0.wait()
    i1.wait()

    def blk_body(blk, carry):
        sl = blk % 2
        nsl = (blk + 1) % 2
        nbase = wid * CPT + (blk + 1) * IB

        @pl.when(blk + 1 < NBLK)
        def _prefetch():
            pltpu.async_copy(edges_h.at[0, pl.ds(nbase, IB)], srcb.at[nsl],
                             sem_i)
            pltpu.async_copy(edges_h.at[1, pl.ds(nbase, IB)], dstb.at[nsl],
                             sem_i)

        src = srcb.at[sl]
        dst = dstb.at[sl]
        g = [None] * IB
        sc = [None] * IB
        dg = [None] * IB
        g[0] = pltpu.async_copy(feat_h.at[src.at[0]], rows.at[0], sem_g)
        for j in range(IB):
            b = j % NBUF
            nb = (j + 1) % NBUF
            if j + 1 < IB:
                if j + 1 >= NBUF:
                    sc[j + 1 - NBUF].wait()
                g[j + 1] = pltpu.async_copy(feat_h.at[src.at[j + 1]],
                                            rows.at[nb], sem_g)
            g[j].wait()
            sc[j] = pltpu.async_copy(rows.at[b], aggsh.at[dst.at[j]],
                                     sem_s, add=True)
            dg[j] = pltpu.async_copy(onesv, degsh.at[dst.at[j]],
                                     sem_d, add=True)
        for j in range(IB - NBUF, IB):
            sc[j].wait()
        for j in range(IB):
            dg[j].wait()

        @pl.when(blk + 1 < NBLK)
        def _drain_prefetch():
            pltpu.make_async_copy(edges_h.at[0, pl.ds(nbase, IB)],
                                  srcb.at[nsl], sem_i).wait()
            pltpu.make_async_copy(edges_h.at[1, pl.ds(nbase, IB)],
                                  dstb.at[nsl], sem_i).wait()

        return carry

    lax.fori_loop(0, NBLK, blk_body, 0)
    plsc.subcore_barrier()

    wr = [None] * (RPT // WB)
    for k in range(RPT // WB):
        base = s * RPT + k * WB
        b = k % NBUF
        if k >= NBUF:
            wr[k - NBUF].wait()
        pltpu.sync_copy(aggsh.at[pl.ds(base, WB)], rows.at[b])
        wr[k] = pltpu.async_copy(rows.at[b], agg_h.at[c, pl.ds(base, WB)],
                                 sem_g)
    pltpu.sync_copy(degsh.at[pl.ds(s * RPT, RPT)], zvec)
    pltpu.sync_copy(zvec, deg_h.at[c, pl.ds(s * RPT, RPT)])
    for k in range(RPT // WB - NBUF, RPT // WB):
        wr[k].wait()


_sc_call = pl.kernel(
    _sc_body,
    mesh=plsc.VectorSubcoreMesh(core_axis_name="c", subcore_axis_name="s",
                                num_cores=NC),
    out_type=[
        jax.ShapeDtypeStruct((NC, NPAD, D), jnp.float32),
        jax.ShapeDtypeStruct((NC, NPAD), jnp.float32),
    ],
    scratch_types=[
        pltpu.VMEM((2, IB, CHUNK), jnp.int32),
        pltpu.VMEM((2, IB, CHUNK), jnp.int32),
        pltpu.VMEM((NBUF, CHUNK, D), jnp.float32),
        pltpu.VMEM((RPT,), jnp.float32),
        pltpu.VMEM((CHUNK,), jnp.float32),
        pltpu.VMEM_SHARED((NPAD, D), jnp.float32),
        pltpu.VMEM_SHARED((NPAD,), jnp.float32),
        pltpu.SemaphoreType.DMA,
        pltpu.SemaphoreType.DMA,
        pltpu.SemaphoreType.DMA,
        pltpu.SemaphoreType.DMA,
    ],
)


BN = 2560


def _tc_body(a_ref, d_ref, w_ref, b_ref, o_ref):
    agg = a_ref[0]
    deg = d_ref[0]
    for c in range(1, NC):
        agg = agg + a_ref[c]
        deg = deg + d_ref[c]
    deg = jnp.maximum(deg, 1.0)
    rst = agg / deg[:, None]
    o_ref[...] = jnp.dot(rst, w_ref[...],
                         preferred_element_type=jnp.float32) + b_ref[...]


_tc_call = pl.pallas_call(
    _tc_body,
    grid=(NPAD // BN,),
    in_specs=[
        pl.BlockSpec((NC, BN, D), lambda i: (0, i, 0)),
        pl.BlockSpec((NC, BN), lambda i: (0, i)),
        pl.BlockSpec((D, D), lambda i: (0, 0)),
        pl.BlockSpec((1, D), lambda i: (0, 0)),
    ],
    out_specs=pl.BlockSpec((BN, D), lambda i: (i, 0)),
    out_shape=jax.ShapeDtypeStruct((N, D), jnp.float32),
)


def kernel(feat, edge_index, W, b):
    pad = EPAD - E
    ar = jnp.arange(pad, dtype=jnp.int32)
    pad_src = ar % N
    pad_dst = N + (ar % (NPAD - N))
    ei = jnp.concatenate(
        [edge_index.astype(jnp.int32), jnp.stack([pad_src, pad_dst])], axis=1)
    ei = ei.reshape(2, EPAD // CHUNK, CHUNK)
    agg2, degp = _sc_call(feat, ei)
    return _tc_call(agg2, degp, W, b.reshape(1, D))
